# baseline (speedup 1.0000x reference)
import functools

import jax
import jax.numpy as jnp
from jax import lax
from jax.experimental import pallas as pl
from jax.experimental.pallas import tpu as pltpu
from jax.experimental.pallas import tpu_sc as plsc

B, S, D = 4, 4096, 1024
NC, NS = 2, 16
NW = NC * NS
ROWS_PER_W = S // NW
LANES = 16
DV = D // LANES
_SHIFT_DV = DV.bit_length() - 1

_sc_mesh = plsc.VectorSubcoreMesh(core_axis_name="c", subcore_axis_name="s")


def _make_sc_add(nb: int, b0: int, c_rows: int):
    nch = ROWS_PER_W // c_rows

    @functools.partial(
        pl.kernel,
        out_type=jax.ShapeDtypeStruct((nb, S, D), jnp.float32),
        mesh=_sc_mesh,
        scratch_types=[
            pltpu.VMEM((3, c_rows, D), jnp.float32),
            pltpu.VMEM((3, nb, c_rows, D), jnp.float32),
            pltpu.SemaphoreType.DMA,
            pltpu.SemaphoreType.DMA,
            pltpu.SemaphoreType.DMA,
            pltpu.SemaphoreType.DMA,
            pltpu.SemaphoreType.DMA,
            pltpu.SemaphoreType.DMA,
        ],
    )
    def sc_add(x_hbm, emb_hbm, out_hbm, ebuf, xbuf,
               isem0, isem1, isem2, osem0, osem1, osem2):
        cid = lax.axis_index("c")
        sid = lax.axis_index("s")
        wid = sid * NC + cid
        row_base = wid * ROWS_PER_W
        isems = (isem0, isem1, isem2)
        osems = (osem0, osem1, osem2)

        def in_cps(k, p):
            r0 = row_base + k * c_rows
            return [
                pltpu.make_async_copy(
                    emb_hbm.at[pl.ds(r0, c_rows)], ebuf.at[p], isems[p]),
                pltpu.make_async_copy(
                    x_hbm.at[pl.ds(b0, nb), pl.ds(r0, c_rows)], xbuf.at[p],
                    isems[p]),
            ]

        def out_cp(k, p):
            r0 = row_base + k * c_rows
            return pltpu.make_async_copy(
                xbuf.at[p], out_hbm.at[:, pl.ds(r0, c_rows)], osems[p])

        def compute(p):
            @plsc.parallel_loop(0, c_rows * DV, unroll=8)
            def _(i):
                r = lax.shift_right_logical(i, _SHIFT_DV)
                sl = pl.ds((i & (DV - 1)) * LANES, LANES)
                e = ebuf[p, r, sl]
                for b in range(nb):
                    xbuf[p, b, r, sl] = xbuf[p, b, r, sl] + e

        for cp in in_cps(0, 0):
            cp.start()
        for k in range(nch):
            p = k % 3
            nxt = k + 1
            if nxt < nch:
                q = nxt % 3
                if k >= 2:
                    out_cp(k - 2, q).wait()
                for cp in in_cps(nxt, q):
                    cp.start()
            for cp in in_cps(k, p):
                cp.wait()
            compute(p)
            out_cp(k, p).start()
        for j in range(max(0, nch - 3), nch):
            out_cp(j, j % 3).wait()

    return sc_add


_sc_add_full = _make_sc_add(B, 0, 8)


def kernel(x, emb_table):
    return _sc_add_full(x, emb_table)
